# baseline (speedup 1.0000x reference)
import functools

import jax
import jax.numpy as jnp
from jax import lax
from jax.experimental import pallas as pl
from jax.experimental.pallas import tpu as pltpu

K = 19
_BF = jnp.bfloat16


def _labels_of(sm):
    best = sm[0]
    arg = jnp.zeros(best.shape, dtype=jnp.int32)
    for k in range(1, K):
        v = sm[k]
        gt = v > best
        best = jnp.where(gt, v, best)
        arg = jnp.where(gt, k, arg)
    return arg


def _fused_kernel(T, sf_ref, tf_ref, ssm_ref, tsm_ref, sw_ref, tw_ref,
                  out_ref, num_ref, aux_ref, slab_ref, tlab_ref,
                  cen_ref, c2_ref, sd_ref, td_ref):
    p = pl.program_id(0)
    i = pl.program_id(1)
    steps = pl.num_programs(1)
    C, hb, w = sf_ref.shape
    r0 = i * hb

    @pl.when((p == 0) & (i == 0))
    def _():
        num_ref[...] = jnp.zeros_like(num_ref)
        aux_ref[...] = jnp.zeros_like(aux_ref)
        sd_ref[...] = jnp.zeros_like(sd_ref)
        td_ref[...] = jnp.zeros_like(td_ref)

    @pl.when(p == 0)
    def _pass1():
        s_arg = _labels_of(ssm_ref[...])
        t_arg = _labels_of(tsm_ref[...])
        slab_ref[pl.ds(r0, hb), :] = s_arg
        tlab_ref[pl.ds(r0, hb), :] = t_arg

        iota = lax.broadcasted_iota(jnp.int32, (K, hb, w), 0)
        s_oh = (iota == s_arg[None]).astype(jnp.float32)
        t_oh = (iota == t_arg[None]).astype(jnp.float32)
        ms3 = s_oh * sw_ref[...].astype(jnp.float32)
        mt3 = t_oh * (1.0 - tw_ref[...].astype(jnp.float32))
        fs = sf_ref[...].astype(_BF).reshape(C, hb * w)
        ft = tf_ref[...].astype(_BF).reshape(C, hb * w)
        ms = ms3.astype(_BF).reshape(K, hb * w)
        mt = mt3.astype(_BF).reshape(K, hb * w)

        dn = (((1,), (1,)), ((), ()))
        contrib = lax.dot_general(fs, ms, dn,
                                  preferred_element_type=jnp.float32)
        contrib += lax.dot_general(ft, mt, dn,
                                   preferred_element_type=jnp.float32)

        def colsum(x):
            return jnp.sum(jnp.sum(x, axis=2), axis=1, keepdims=True)

        num_ref[...] += contrib
        aux_ref[...] += jnp.concatenate(
            [colsum(ms3), colsum(mt3), colsum(s_oh), colsum(t_oh)], axis=1)

    @pl.when((p == 1) & (i == 0))
    def _centroids():
        den = aux_ref[:, 0:1] + aux_ref[:, 1:2]
        cen = num_ref[...].T / jnp.maximum(den, 1.0)
        cen_ref[...] = cen.astype(_BF)
        c2_ref[...] = jnp.sum(cen * cen, axis=1, keepdims=True)

    @pl.when(p == 1)
    def _pass2():
        cen_bf = cen_ref[...]
        c2 = c2_ref[...]
        ones_row = jnp.ones((1, C), dtype=_BF)

        def seg_d(f3, lab):
            f = f3.astype(_BF).reshape(C, hb * w)
            fc = lax.dot_general(cen_bf, f, (((1,), (0,)), ((), ())),
                                 preferred_element_type=jnp.float32)
            f2 = lax.dot_general(ones_row, f * f, (((1,), (0,)), ((), ())),
                                 preferred_element_type=jnp.float32)
            d2 = jnp.maximum(f2 + c2 - 2.0 * fc, 0.0)
            dist = jnp.sqrt(d2 + 1e-12) * (1.0 / C)
            oh = (lax.broadcasted_iota(jnp.int32, (K, hb, w), 0) == lab[None])
            oh = oh.astype(_BF).reshape(K, hb * w)
            return lax.dot_general(dist.astype(_BF), oh,
                                   (((1,), (1,)), ((), ())),
                                   preferred_element_type=jnp.float32)

        sd_ref[...] += seg_d(sf_ref[...], slab_ref[pl.ds(r0, hb), :])
        td_ref[...] += seg_d(tf_ref[...], tlab_ref[pl.ds(r0, hb), :])

    @pl.when((p == 1) & (i == steps - 1))
    def _finalize():
        den = aux_ref[:, 0:1] + aux_ref[:, 1:2]
        cen_valid = den > 0.0
        row_i = lax.broadcasted_iota(jnp.int32, (K, K), 0)
        col_j = lax.broadcasted_iota(jnp.int32, (K, K), 1)
        eye = row_i == col_j
        inf = jnp.float32(jnp.inf)

        def terms(segd, counts):
            mean_pc = segd.T / jnp.maximum(counts, 1.0)
            valid = (counts > 0.0) & cen_valid
            diag = jnp.sum(jnp.where(eye, mean_pc, 0.0), axis=1,
                           keepdims=True)
            nvalid = jnp.sum(valid.astype(jnp.float32))
            intra = jnp.sum(jnp.where(valid, diag, 0.0)) / jnp.maximum(
                nvalid, 1.0)
            off = jnp.where(eye | (~cen_valid).reshape(1, K), -inf, mean_pc)
            per_i_max = jnp.max(off, axis=1, keepdims=True)
            inter_min = jnp.min(jnp.where(valid, per_i_max, inf))
            return intra, inter_min

        intra_s, inter_s = terms(sd_ref[...], aux_ref[:, 2:3])
        intra_t, inter_t = terms(td_ref[...], aux_ref[:, 3:4])
        loss = intra_s + intra_t - 0.1 * (inter_s + inter_t)
        out_ref[...] = jnp.full((1, 1), loss, dtype=jnp.float32)


@functools.partial(jax.jit, static_argnames=("hb",))
def _run(sf, ssm, scon, tf, tsm, tcon, hb=16):
    B, C, h, w = sf.shape
    T = h // hb
    steps = B * T

    sf3 = sf.reshape(B * C, h, w)
    tf3 = tf.reshape(B * C, h, w)
    ssm3 = ssm.reshape(B * K, h, w)
    tsm3 = tsm.reshape(B * K, h, w)

    def fmap(p, i):
        return (i // T, i % T, 0)

    def smap(p, i):
        z = jnp.where(p == 0, 1, 0)
        return ((i // T) * z, (i % T) * z, 0)

    feat_spec = pl.BlockSpec((C, hb, w), fmap)
    sm_spec = pl.BlockSpec((K, hb, w), smap)
    w_spec = pl.BlockSpec((1, hb, w), smap)

    out = pl.pallas_call(
        functools.partial(_fused_kernel, T),
        grid=(2, steps),
        in_specs=[feat_spec, feat_spec, sm_spec, sm_spec, w_spec, w_spec],
        out_specs=pl.BlockSpec((1, 1), lambda p, i: (0, 0)),
        out_shape=jax.ShapeDtypeStruct((1, 1), jnp.float32),
        scratch_shapes=[
            pltpu.VMEM((C, K), jnp.float32),
            pltpu.VMEM((K, 4), jnp.float32),
            pltpu.VMEM((B * h, w), jnp.int32),
            pltpu.VMEM((B * h, w), jnp.int32),
            pltpu.VMEM((K, C), _BF),
            pltpu.VMEM((K, 1), jnp.float32),
            pltpu.VMEM((K, K), jnp.float32),
            pltpu.VMEM((K, K), jnp.float32),
        ],
    )(sf3, tf3, ssm3, tsm3, scon, tcon)
    return out.reshape(())


def kernel(source_feat, source_softmax, source_confidence,
           target_feat, target_softmax, target_confidence):
    return _run(source_feat, source_softmax, source_confidence,
                target_feat, target_softmax, target_confidence)


# speedup vs baseline: 1.0208x; 1.0208x over previous
import functools

import jax
import jax.numpy as jnp
from jax import lax
from jax.experimental import pallas as pl
from jax.experimental.pallas import tpu as pltpu

K = 19
_BF = jnp.bfloat16


def _labels_of(sm):
    best = sm[0]
    arg = jnp.zeros(best.shape, dtype=jnp.int32)
    for k in range(1, K):
        v = sm[k]
        gt = v > best
        best = jnp.where(gt, v, best)
        arg = jnp.where(gt, k, arg)
    return arg


def _fused_kernel(T, sf_ref, tf_ref, ssm_ref, tsm_ref, sw_ref, tw_ref,
                  out_ref, num_ref, aux_ref, slab_ref, tlab_ref,
                  cen_ref, c2_ref, sd_ref, td_ref):
    p = pl.program_id(0)
    i = pl.program_id(1)
    steps = pl.num_programs(1)
    C, hb, w = sf_ref.shape
    r0 = i * hb

    @pl.when((p == 0) & (i == 0))
    def _():
        num_ref[...] = jnp.zeros_like(num_ref)
        aux_ref[...] = jnp.zeros_like(aux_ref)
        sd_ref[...] = jnp.zeros_like(sd_ref)
        td_ref[...] = jnp.zeros_like(td_ref)

    @pl.when(p == 0)
    def _pass1():
        s_arg = _labels_of(ssm_ref[...])
        t_arg = _labels_of(tsm_ref[...])
        slab_ref[pl.ds(r0, hb), :] = s_arg
        tlab_ref[pl.ds(r0, hb), :] = t_arg

        iota = lax.broadcasted_iota(jnp.int32, (K, hb, w), 0)
        s_oh = (iota == s_arg[None]).astype(jnp.float32)
        t_oh = (iota == t_arg[None]).astype(jnp.float32)
        ms3 = s_oh * sw_ref[...].astype(jnp.float32)
        mt3 = t_oh * (1.0 - tw_ref[...].astype(jnp.float32))
        fs = sf_ref[...].astype(_BF).reshape(C, hb * w)
        ft = tf_ref[...].astype(_BF).reshape(C, hb * w)
        ms = ms3.astype(_BF).reshape(K, hb * w)
        mt = mt3.astype(_BF).reshape(K, hb * w)

        dn = (((1,), (1,)), ((), ()))
        contrib = lax.dot_general(fs, ms, dn,
                                  preferred_element_type=jnp.float32)
        contrib += lax.dot_general(ft, mt, dn,
                                   preferred_element_type=jnp.float32)

        def colsum(x):
            return jnp.sum(jnp.sum(x, axis=2), axis=1, keepdims=True)

        num_ref[...] += contrib
        aux_ref[...] += jnp.concatenate(
            [colsum(ms3), colsum(mt3), colsum(s_oh), colsum(t_oh)], axis=1)

    @pl.when((p == 1) & (i == 0))
    def _centroids():
        den = aux_ref[:, 0:1] + aux_ref[:, 1:2]
        cen = num_ref[...].T / jnp.maximum(den, 1.0)
        cen_ref[...] = cen.astype(_BF)
        c2_ref[...] = jnp.sum(cen * cen, axis=1, keepdims=True)

    @pl.when(p == 1)
    def _pass2():
        cen_bf = cen_ref[...]
        c2 = c2_ref[...]
        ones_row = jnp.ones((1, C), dtype=_BF)

        def seg_d(f3, lab):
            f = f3.astype(_BF).reshape(C, hb * w)
            fc = lax.dot_general(cen_bf, f, (((1,), (0,)), ((), ())),
                                 preferred_element_type=jnp.float32)
            f2 = lax.dot_general(ones_row, f * f, (((1,), (0,)), ((), ())),
                                 preferred_element_type=jnp.float32)
            d2 = jnp.maximum(f2 + c2 - 2.0 * fc, 0.0)
            dist = jnp.sqrt(d2 + 1e-12) * (1.0 / C)
            oh = (lax.broadcasted_iota(jnp.int32, (K, hb, w), 0) == lab[None])
            oh = oh.astype(_BF).reshape(K, hb * w)
            return lax.dot_general(dist.astype(_BF), oh,
                                   (((1,), (1,)), ((), ())),
                                   preferred_element_type=jnp.float32)

        sd_ref[...] += seg_d(sf_ref[...], slab_ref[pl.ds(r0, hb), :])
        td_ref[...] += seg_d(tf_ref[...], tlab_ref[pl.ds(r0, hb), :])

    @pl.when((p == 1) & (i == steps - 1))
    def _finalize():
        den = aux_ref[:, 0:1] + aux_ref[:, 1:2]
        cen_valid = den > 0.0
        row_i = lax.broadcasted_iota(jnp.int32, (K, K), 0)
        col_j = lax.broadcasted_iota(jnp.int32, (K, K), 1)
        eye = row_i == col_j
        inf = jnp.float32(jnp.inf)

        def terms(segd, counts):
            mean_pc = segd.T / jnp.maximum(counts, 1.0)
            valid = (counts > 0.0) & cen_valid
            diag = jnp.sum(jnp.where(eye, mean_pc, 0.0), axis=1,
                           keepdims=True)
            nvalid = jnp.sum(valid.astype(jnp.float32))
            intra = jnp.sum(jnp.where(valid, diag, 0.0)) / jnp.maximum(
                nvalid, 1.0)
            off = jnp.where(eye | (~cen_valid).reshape(1, K), -inf, mean_pc)
            per_i_max = jnp.max(off, axis=1, keepdims=True)
            inter_min = jnp.min(jnp.where(valid, per_i_max, inf))
            return intra, inter_min

        intra_s, inter_s = terms(sd_ref[...], aux_ref[:, 2:3])
        intra_t, inter_t = terms(td_ref[...], aux_ref[:, 3:4])
        loss = intra_s + intra_t - 0.1 * (inter_s + inter_t)
        out_ref[...] = jnp.full((1, 1), loss, dtype=jnp.float32)


@functools.partial(jax.jit, static_argnames=("hb",))
def _run(sf, ssm, scon, tf, tsm, tcon, hb=32):
    B, C, h, w = sf.shape
    T = h // hb
    steps = B * T

    sf3 = sf.reshape(B * C, h, w)
    tf3 = tf.reshape(B * C, h, w)
    ssm3 = ssm.reshape(B * K, h, w)
    tsm3 = tsm.reshape(B * K, h, w)

    def fmap(p, i):
        return (i // T, i % T, 0)

    def smap(p, i):
        z = jnp.where(p == 0, 1, 0)
        return ((i // T) * z, (i % T) * z, 0)

    feat_spec = pl.BlockSpec((C, hb, w), fmap)
    sm_spec = pl.BlockSpec((K, hb, w), smap)
    w_spec = pl.BlockSpec((1, hb, w), smap)

    out = pl.pallas_call(
        functools.partial(_fused_kernel, T),
        grid=(2, steps),
        in_specs=[feat_spec, feat_spec, sm_spec, sm_spec, w_spec, w_spec],
        out_specs=pl.BlockSpec((1, 1), lambda p, i: (0, 0)),
        out_shape=jax.ShapeDtypeStruct((1, 1), jnp.float32),
        scratch_shapes=[
            pltpu.VMEM((C, K), jnp.float32),
            pltpu.VMEM((K, 4), jnp.float32),
            pltpu.VMEM((B * h, w), jnp.int32),
            pltpu.VMEM((B * h, w), jnp.int32),
            pltpu.VMEM((K, C), _BF),
            pltpu.VMEM((K, 1), jnp.float32),
            pltpu.VMEM((K, K), jnp.float32),
            pltpu.VMEM((K, K), jnp.float32),
        ],
    )(sf3, tf3, ssm3, tsm3, scon, tcon)
    return out.reshape(())


def kernel(source_feat, source_softmax, source_confidence,
           target_feat, target_softmax, target_confidence):
    return _run(source_feat, source_softmax, source_confidence,
                target_feat, target_softmax, target_confidence)


# speedup vs baseline: 1.0366x; 1.0154x over previous
import functools

import jax
import jax.numpy as jnp
from jax import lax
from jax.experimental import pallas as pl
from jax.experimental.pallas import tpu as pltpu

K = 19
_BF = jnp.bfloat16


def _labels_of(sm):
    best = sm[0]
    arg = jnp.zeros(best.shape, dtype=jnp.int32)
    for k in range(1, K):
        v = sm[k]
        gt = v > best
        best = jnp.where(gt, v, best)
        arg = jnp.where(gt, k, arg)
    return arg


def _fused_kernel(T, sf_ref, tf_ref, ssm_ref, tsm_ref, sw_ref, tw_ref,
                  out_ref, num_ref, aux_ref, slab_ref, tlab_ref,
                  cen_ref, c2_ref, sd_ref, td_ref):
    p = pl.program_id(0)
    iraw = pl.program_id(1)
    steps = pl.num_programs(1)
    i = jnp.where(p == 0, iraw, steps - 1 - iraw)
    C, hb, w = sf_ref.shape
    r0 = i * hb

    @pl.when((p == 0) & (i == 0))
    def _():
        num_ref[...] = jnp.zeros_like(num_ref)
        aux_ref[...] = jnp.zeros_like(aux_ref)
        sd_ref[...] = jnp.zeros_like(sd_ref)
        td_ref[...] = jnp.zeros_like(td_ref)

    @pl.when(p == 0)
    def _pass1():
        s_arg = _labels_of(ssm_ref[...])
        t_arg = _labels_of(tsm_ref[...])
        slab_ref[pl.ds(r0, hb), :] = s_arg
        tlab_ref[pl.ds(r0, hb), :] = t_arg

        iota = lax.broadcasted_iota(jnp.int32, (K, hb, w), 0)
        s_oh = (iota == s_arg[None]).astype(jnp.float32)
        t_oh = (iota == t_arg[None]).astype(jnp.float32)
        ms3 = s_oh * sw_ref[...].astype(jnp.float32)
        mt3 = t_oh * (1.0 - tw_ref[...].astype(jnp.float32))
        fs = sf_ref[...].astype(_BF).reshape(C, hb * w)
        ft = tf_ref[...].astype(_BF).reshape(C, hb * w)
        ms = ms3.astype(_BF).reshape(K, hb * w)
        mt = mt3.astype(_BF).reshape(K, hb * w)

        dn = (((1,), (1,)), ((), ()))
        contrib = lax.dot_general(fs, ms, dn,
                                  preferred_element_type=jnp.float32)
        contrib += lax.dot_general(ft, mt, dn,
                                   preferred_element_type=jnp.float32)

        def colsum(x):
            return jnp.sum(jnp.sum(x, axis=2), axis=1, keepdims=True)

        num_ref[...] += contrib
        aux_ref[...] += jnp.concatenate(
            [colsum(ms3), colsum(mt3), colsum(s_oh), colsum(t_oh)], axis=1)

    @pl.when((p == 1) & (iraw == 0))
    def _centroids():
        den = aux_ref[:, 0:1] + aux_ref[:, 1:2]
        cen = num_ref[...].T / jnp.maximum(den, 1.0)
        cen_ref[...] = cen.astype(_BF)
        c2_ref[...] = jnp.sum(cen * cen, axis=1, keepdims=True)

    @pl.when(p == 1)
    def _pass2():
        cen_bf = cen_ref[...]
        c2 = c2_ref[...]
        ones_row = jnp.ones((1, C), dtype=_BF)

        def seg_d(f3, lab):
            f = f3.astype(_BF).reshape(C, hb * w)
            fc = lax.dot_general(cen_bf, f, (((1,), (0,)), ((), ())),
                                 preferred_element_type=jnp.float32)
            f2 = lax.dot_general(ones_row, f * f, (((1,), (0,)), ((), ())),
                                 preferred_element_type=jnp.float32)
            d2 = jnp.maximum(f2 + c2 - 2.0 * fc, 0.0)
            dist = jnp.sqrt(d2 + 1e-12) * (1.0 / C)
            oh = (lax.broadcasted_iota(jnp.int32, (K, hb, w), 0) == lab[None])
            oh = oh.astype(_BF).reshape(K, hb * w)
            return lax.dot_general(dist.astype(_BF), oh,
                                   (((1,), (1,)), ((), ())),
                                   preferred_element_type=jnp.float32)

        sd_ref[...] += seg_d(sf_ref[...], slab_ref[pl.ds(r0, hb), :])
        td_ref[...] += seg_d(tf_ref[...], tlab_ref[pl.ds(r0, hb), :])

    @pl.when((p == 1) & (iraw == steps - 1))
    def _finalize():
        den = aux_ref[:, 0:1] + aux_ref[:, 1:2]
        cen_valid = den > 0.0
        row_i = lax.broadcasted_iota(jnp.int32, (K, K), 0)
        col_j = lax.broadcasted_iota(jnp.int32, (K, K), 1)
        eye = row_i == col_j
        inf = jnp.float32(jnp.inf)

        def terms(segd, counts):
            mean_pc = segd.T / jnp.maximum(counts, 1.0)
            valid = (counts > 0.0) & cen_valid
            diag = jnp.sum(jnp.where(eye, mean_pc, 0.0), axis=1,
                           keepdims=True)
            nvalid = jnp.sum(valid.astype(jnp.float32))
            intra = jnp.sum(jnp.where(valid, diag, 0.0)) / jnp.maximum(
                nvalid, 1.0)
            off = jnp.where(eye | (~cen_valid).reshape(1, K), -inf, mean_pc)
            per_i_max = jnp.max(off, axis=1, keepdims=True)
            inter_min = jnp.min(jnp.where(valid, per_i_max, inf))
            return intra, inter_min

        intra_s, inter_s = terms(sd_ref[...], aux_ref[:, 2:3])
        intra_t, inter_t = terms(td_ref[...], aux_ref[:, 3:4])
        loss = intra_s + intra_t - 0.1 * (inter_s + inter_t)
        out_ref[...] = jnp.full((1, 1), loss, dtype=jnp.float32)


@functools.partial(jax.jit, static_argnames=("hb",))
def _run(sf, ssm, scon, tf, tsm, tcon, hb=32):
    B, C, h, w = sf.shape
    T = h // hb
    steps = B * T

    sf3 = sf.reshape(B * C, h, w)
    tf3 = tf.reshape(B * C, h, w)
    ssm3 = ssm.reshape(B * K, h, w)
    tsm3 = tsm.reshape(B * K, h, w)

    def fmap(p, i):
        j = jnp.where(p == 0, i, steps - 1 - i)
        return (j // T, j % T, 0)

    def smap(p, i):
        j = jnp.where(p == 0, i, steps - 1)
        return (j // T, j % T, 0)

    feat_spec = pl.BlockSpec((C, hb, w), fmap)
    sm_spec = pl.BlockSpec((K, hb, w), smap)
    w_spec = pl.BlockSpec((1, hb, w), smap)

    out = pl.pallas_call(
        functools.partial(_fused_kernel, T),
        grid=(2, steps),
        in_specs=[feat_spec, feat_spec, sm_spec, sm_spec, w_spec, w_spec],
        out_specs=pl.BlockSpec((1, 1), lambda p, i: (0, 0)),
        out_shape=jax.ShapeDtypeStruct((1, 1), jnp.float32),
        scratch_shapes=[
            pltpu.VMEM((C, K), jnp.float32),
            pltpu.VMEM((K, 4), jnp.float32),
            pltpu.VMEM((B * h, w), jnp.int32),
            pltpu.VMEM((B * h, w), jnp.int32),
            pltpu.VMEM((K, C), _BF),
            pltpu.VMEM((K, 1), jnp.float32),
            pltpu.VMEM((K, K), jnp.float32),
            pltpu.VMEM((K, K), jnp.float32),
        ],
    )(sf3, tf3, ssm3, tsm3, scon, tcon)
    return out.reshape(())


def kernel(source_feat, source_softmax, source_confidence,
           target_feat, target_softmax, target_confidence):
    return _run(source_feat, source_softmax, source_confidence,
                target_feat, target_softmax, target_confidence)
